# Initial kernel scaffold; baseline (speedup 1.0000x reference)
import jax
import jax.numpy as jnp
from jax.experimental import pallas as pl


def kernel(features, emb_table, W1, b1, W2, b2, W3, b3):
    raise NotImplementedError("write your pallas kernel here")


# speedup vs baseline: 13.7051x; 13.7051x over previous
import functools

import jax
import jax.numpy as jnp
from jax import lax
from jax.experimental import pallas as pl
from jax.experimental.pallas import tpu as pltpu
from jax.experimental.pallas import tpu_sc as plsc

_NC = 2
_NS = 16
_NW = _NC * _NS

_CH = 128


def _sc_gather(table, idx3):
    nw, n_ch, ch = idx3.shape
    d = table.shape[1]
    b_per_w = n_ch * ch
    n_rows = nw * b_per_w

    mesh = plsc.VectorSubcoreMesh(
        core_axis_name="c", subcore_axis_name="s",
        num_cores=_NC, num_subcores=_NS)

    @functools.partial(
        pl.kernel,
        mesh=mesh,
        out_type=jax.ShapeDtypeStruct((n_rows, d), jnp.float32),
        scratch_types=[
            pltpu.VMEM((n_ch, ch), jnp.int32),
            pltpu.VMEM((ch, d), jnp.float32),
            pltpu.VMEM((ch, d), jnp.float32),
            pltpu.SemaphoreType.DMA,
            pltpu.SemaphoreType.DMA,
        ],
    )
    def k(table_hbm, idx_hbm, out_hbm, idx_v, buf0, buf1, sem0, sem1):
        wid = lax.axis_index("s") * _NC + lax.axis_index("c")
        base = wid * b_per_w
        pltpu.sync_copy(idx_hbm.at[wid], idx_v)

        def start(j, buf, sem):
            pltpu.async_copy(table_hbm.at[idx_v.at[j]], buf, sem)

        def wait(j, buf, sem):
            pltpu.make_async_copy(table_hbm.at[idx_v.at[j]], buf, sem).wait()

        def drain(j, buf):
            pltpu.sync_copy(buf, out_hbm.at[pl.ds(base + j * ch, ch)])

        start(0, buf0, sem0)

        def body(jj, carry):
            j = 2 * jj
            start(j + 1, buf1, sem1)
            wait(j, buf0, sem0)
            drain(j, buf0)

            @pl.when(j + 2 < n_ch)
            def _():
                start(j + 2, buf0, sem0)

            wait(j + 1, buf1, sem1)
            drain(j + 1, buf1)
            return carry

        lax.fori_loop(0, n_ch // 2, body, 0)

    return k(table, idx3)


def _tc_mlp(x, w1_bf16, b1, w2, b2, w3, b3, tb=256, tk=2048):
    batch, kdim = x.shape
    u1 = w1_bf16.shape[1]
    u2 = w2.shape[1]
    nb = batch // tb
    nk = kdim // tk

    def body(x_ref, w1_ref, b1_ref, w2_ref, b2_ref, w3_ref, b3_ref,
             out_ref, acc_ref):
        k = pl.program_id(0)
        i = pl.program_id(1)
        part = jnp.dot(x_ref[...].astype(jnp.bfloat16), w1_ref[...],
                       preferred_element_type=jnp.float32)
        sl = pl.ds(i * tb, tb)

        @pl.when(k == 0)
        def _():
            acc_ref[sl, :] = part

        @pl.when(k > 0)
        def _():
            acc_ref[sl, :] += part

        @pl.when(k == nk - 1)
        def _():
            h1 = jnp.maximum(acc_ref[sl, :] + b1_ref[...], 0.0)
            h2 = jnp.maximum(
                jnp.dot(h1, w2_ref[...], preferred_element_type=jnp.float32)
                + b2_ref[...], 0.0)
            z = (jnp.dot(h2, w3_ref[...], preferred_element_type=jnp.float32)
                 + b3_ref[...])
            out_ref[...] = jax.nn.sigmoid(z)

    return pl.pallas_call(
        body,
        grid=(nk, nb),
        in_specs=[
            pl.BlockSpec((tb, tk), lambda k, i: (i, k)),
            pl.BlockSpec((tk, u1), lambda k, i: (k, 0)),
            pl.BlockSpec((1, u1), lambda k, i: (0, 0)),
            pl.BlockSpec((u1, u2), lambda k, i: (0, 0)),
            pl.BlockSpec((1, u2), lambda k, i: (0, 0)),
            pl.BlockSpec((u2, 1), lambda k, i: (0, 0)),
            pl.BlockSpec((1, 1), lambda k, i: (0, 0)),
        ],
        out_specs=pl.BlockSpec((tb, 1), lambda k, i: (i, 0)),
        out_shape=jax.ShapeDtypeStruct((batch, 1), jnp.float32),
        scratch_shapes=[pltpu.VMEM((batch, u1), jnp.float32)],
        compiler_params=pltpu.CompilerParams(
            dimension_semantics=("arbitrary", "arbitrary")),
    )(x, w1_bf16, b1, w2, b2, w3, b3)


def kernel(features, emb_table, W1, b1, W2, b2, W3, b3):
    batch, seq = features.shape
    d = emb_table.shape[1]
    idx3 = features.astype(jnp.int32).reshape(_NW, -1, _CH)
    rows = _sc_gather(emb_table, idx3)
    x = rows.reshape(batch, seq * d)
    return _tc_mlp(
        x,
        W1.astype(jnp.bfloat16),
        b1.reshape(1, -1),
        W2,
        b2.reshape(1, -1),
        W3,
        b3.reshape(1, 1),
    )


# speedup vs baseline: 21.2249x; 1.5487x over previous
import functools

import jax
import jax.numpy as jnp
from jax import lax
from jax.experimental import pallas as pl
from jax.experimental.pallas import tpu as pltpu
from jax.experimental.pallas import tpu_sc as plsc

_NC = 2
_NS = 16
_NW = _NC * _NS

_CH = 128


def _sc_gather(table, idx3):
    nw, n_ch, ch = idx3.shape
    d = table.shape[1]
    b_per_w = n_ch * ch
    n_rows = nw * b_per_w

    mesh = plsc.VectorSubcoreMesh(
        core_axis_name="c", subcore_axis_name="s",
        num_cores=_NC, num_subcores=_NS)

    @functools.partial(
        pl.kernel,
        mesh=mesh,
        out_type=jax.ShapeDtypeStruct((n_rows, d), jnp.float32),
        scratch_types=[
            pltpu.VMEM((n_ch, ch), jnp.int32),
            pltpu.VMEM((ch, d), jnp.float32),
            pltpu.VMEM((ch, d), jnp.float32),
            pltpu.SemaphoreType.DMA,
            pltpu.SemaphoreType.DMA,
        ],
    )
    def k(table_hbm, idx_hbm, out_hbm, idx_v, buf0, buf1, sem0, sem1):
        wid = lax.axis_index("s") * _NC + lax.axis_index("c")
        base = wid * b_per_w
        pltpu.sync_copy(idx_hbm.at[wid], idx_v)

        def start(j, buf, sem):
            pltpu.async_copy(table_hbm.at[idx_v.at[j]], buf, sem)

        def wait(j, buf, sem):
            pltpu.make_async_copy(table_hbm.at[idx_v.at[j]], buf, sem).wait()

        def drain(j, buf):
            pltpu.sync_copy(buf, out_hbm.at[pl.ds(base + j * ch, ch)])

        start(0, buf0, sem0)

        def body(jj, carry):
            j = 2 * jj
            start(j + 1, buf1, sem1)
            wait(j, buf0, sem0)
            drain(j, buf0)

            @pl.when(j + 2 < n_ch)
            def _():
                start(j + 2, buf0, sem0)

            wait(j + 1, buf1, sem1)
            drain(j + 1, buf1)
            return carry

        lax.fori_loop(0, n_ch // 2, body, 0)

    return k(table, idx3)


def _tc_mlp(x3, w1, b1, w2, b2, w3, b3, tb=256, lg=16):
    batch, seq, d = x3.shape
    u1 = w1.shape[1]
    u2 = w2.shape[1]
    nb = batch // tb
    nk = seq // lg

    def body(x_ref, w1_ref, b1_ref, w2_ref, b2_ref, w3_ref, b3_ref,
             out_ref, acc_ref):
        k = pl.program_id(0)
        i = pl.program_id(1)
        xb = jnp.concatenate(
            [x_ref[:, j, :] for j in range(lg)], axis=1).astype(jnp.bfloat16)
        part = jnp.dot(xb, w1_ref[...], preferred_element_type=jnp.float32)
        sl = pl.ds(i * tb, tb)

        @pl.when(k == 0)
        def _():
            acc_ref[sl, :] = part

        @pl.when(k > 0)
        def _():
            acc_ref[sl, :] += part

        @pl.when(k == nk - 1)
        def _():
            h1 = jnp.maximum(acc_ref[sl, :] + b1_ref[...], 0.0)
            h2 = jnp.maximum(
                jnp.dot(h1, w2_ref[...], preferred_element_type=jnp.float32)
                + b2_ref[...], 0.0)
            z = (jnp.dot(h2, w3_ref[...], preferred_element_type=jnp.float32)
                 + b3_ref[...])
            out_ref[...] = jax.nn.sigmoid(z)

    return pl.pallas_call(
        body,
        grid=(nk, nb),
        in_specs=[
            pl.BlockSpec((tb, lg, d), lambda k, i: (i, k, 0)),
            pl.BlockSpec((lg * d, u1), lambda k, i: (k, 0)),
            pl.BlockSpec((1, u1), lambda k, i: (0, 0)),
            pl.BlockSpec((u1, u2), lambda k, i: (0, 0)),
            pl.BlockSpec((1, u2), lambda k, i: (0, 0)),
            pl.BlockSpec((u2, 1), lambda k, i: (0, 0)),
            pl.BlockSpec((1, 1), lambda k, i: (0, 0)),
        ],
        out_specs=pl.BlockSpec((tb, 1), lambda k, i: (i, 0)),
        out_shape=jax.ShapeDtypeStruct((batch, 1), jnp.float32),
        scratch_shapes=[pltpu.VMEM((batch, u1), jnp.float32)],
        compiler_params=pltpu.CompilerParams(
            dimension_semantics=("arbitrary", "arbitrary")),
    )(x3, w1, b1, w2, b2, w3, b3)


def kernel(features, emb_table, W1, b1, W2, b2, W3, b3):
    batch, seq = features.shape
    d = emb_table.shape[1]
    idx3 = features.astype(jnp.int32).reshape(_NW, -1, _CH)
    rows = _sc_gather(emb_table, idx3)
    x3 = rows.reshape(batch, seq, d)
    return _tc_mlp(
        x3,
        W1.astype(jnp.bfloat16),
        b1.reshape(1, -1),
        W2,
        b2.reshape(1, -1),
        W3,
        b3.reshape(1, 1),
    )


# speedup vs baseline: 22.7228x; 1.0706x over previous
import functools

import jax
import jax.numpy as jnp
from jax import lax
from jax.experimental import pallas as pl
from jax.experimental.pallas import tpu as pltpu
from jax.experimental.pallas import tpu_sc as plsc

_NC = 2
_NS = 16
_NW = _NC * _NS

_CH = 128


def _sc_gather(table, idx3):
    nw, n_ch, ch = idx3.shape
    d = table.shape[1]
    b_per_w = n_ch * ch
    n_rows = nw * b_per_w

    mesh = plsc.VectorSubcoreMesh(
        core_axis_name="c", subcore_axis_name="s",
        num_cores=_NC, num_subcores=_NS)

    @functools.partial(
        pl.kernel,
        mesh=mesh,
        out_type=jax.ShapeDtypeStruct((n_rows, d), jnp.float32),
        scratch_types=[
            pltpu.VMEM((n_ch, ch), jnp.int32),
            pltpu.VMEM((ch, d), jnp.float32),
            pltpu.VMEM((ch, d), jnp.float32),
            pltpu.SemaphoreType.DMA,
            pltpu.SemaphoreType.DMA,
        ],
    )
    def k(table_hbm, idx_hbm, out_hbm, idx_v, buf0, buf1, sem0, sem1):
        wid = lax.axis_index("s") * _NC + lax.axis_index("c")
        base = wid * b_per_w
        pltpu.sync_copy(idx_hbm.at[wid], idx_v)

        def start(j, buf, sem):
            pltpu.async_copy(table_hbm.at[idx_v.at[j]], buf, sem)

        def wait(j, buf, sem):
            pltpu.make_async_copy(table_hbm.at[idx_v.at[j]], buf, sem).wait()

        def drain(j, buf):
            pltpu.sync_copy(buf, out_hbm.at[pl.ds(base + j * ch, ch)])

        start(0, buf0, sem0)

        def body(jj, carry):
            j = 2 * jj
            start(j + 1, buf1, sem1)
            wait(j, buf0, sem0)
            drain(j, buf0)

            @pl.when(j + 2 < n_ch)
            def _():
                start(j + 2, buf0, sem0)

            wait(j + 1, buf1, sem1)
            drain(j + 1, buf1)
            return carry

        lax.fori_loop(0, n_ch // 2, body, 0)

    return k(table, idx3)


def _tc_mlp(x3, w1, b1, w2, b2, w3, b3, tb=256, lg=16):
    batch, seq, d = x3.shape
    u1 = w1.shape[1]
    u2 = w2.shape[1]
    nb = batch // tb
    nk = seq // lg

    def body(x_ref, w1_ref, b1_ref, w2_ref, b2_ref, w3_ref, b3_ref,
             out_ref, acc_ref):
        k = pl.program_id(0)
        i = pl.program_id(1)
        xb = jnp.concatenate(
            [x_ref[:, j, :] for j in range(lg)], axis=1).astype(jnp.bfloat16)
        part = jnp.dot(xb, w1_ref[...], preferred_element_type=jnp.float32)
        sl = pl.ds(i * tb, tb)

        @pl.when(k == 0)
        def _():
            acc_ref[sl, :] = part

        @pl.when(k > 0)
        def _():
            acc_ref[sl, :] += part

        @pl.when(k == nk - 1)
        def _():
            h1 = jnp.maximum(acc_ref[sl, :] + b1_ref[...], 0.0)
            h2 = jnp.maximum(
                jnp.dot(h1, w2_ref[...], preferred_element_type=jnp.float32)
                + b2_ref[...], 0.0)
            z = (jnp.dot(h2, w3_ref[...], preferred_element_type=jnp.float32)
                 + b3_ref[...])
            out_ref[...] = jax.nn.sigmoid(z)

    return pl.pallas_call(
        body,
        grid=(nk, nb),
        in_specs=[
            pl.BlockSpec((tb, lg, d), lambda k, i: (i, k, 0)),
            pl.BlockSpec((lg * d, u1), lambda k, i: (k, 0)),
            pl.BlockSpec((1, u1), lambda k, i: (0, 0)),
            pl.BlockSpec((u1, u2), lambda k, i: (0, 0)),
            pl.BlockSpec((1, u2), lambda k, i: (0, 0)),
            pl.BlockSpec((u2, 1), lambda k, i: (0, 0)),
            pl.BlockSpec((1, 1), lambda k, i: (0, 0)),
        ],
        out_specs=pl.BlockSpec((tb, 1), lambda k, i: (i, 0)),
        out_shape=jax.ShapeDtypeStruct((batch, 1), jnp.float32),
        scratch_shapes=[pltpu.VMEM((batch, u1), jnp.float32)],
        compiler_params=pltpu.CompilerParams(
            dimension_semantics=("arbitrary", "arbitrary")),
    )(x3, w1, b1, w2, b2, w3, b3)


_CHUNKS = 4


def kernel(features, emb_table, W1, b1, W2, b2, W3, b3):
    batch, seq = features.shape
    d = emb_table.shape[1]
    bc = batch // _CHUNKS
    idx = features.astype(jnp.int32).reshape(_CHUNKS, _NW, -1, _CH)
    w1b = W1.astype(jnp.bfloat16)
    b1r = b1.reshape(1, -1)
    b2r = b2.reshape(1, -1)
    b3r = b3.reshape(1, 1)

    rows = [_sc_gather(emb_table, idx[c]) for c in range(_CHUNKS)]
    outs = [
        _tc_mlp(rows[c].reshape(bc, seq, d), w1b, b1r, W2, b2r, W3, b3r)
        for c in range(_CHUNKS)
    ]
    return jnp.concatenate(outs, axis=0)
